# baseline (speedup 1.0000x reference)
import jax
import jax.numpy as jnp
from jax import lax
from jax.experimental import pallas as pl
from jax.experimental.pallas import tpu as pltpu

B = 128
N = 100000
BC = 2048
NBLK = (N + BC - 1) // BC

_TINY = float(jnp.finfo(jnp.float32).tiny)
_NEG_INF = float("-inf")


def _rol(x, d):
    return lax.shift_left(x, jnp.int32(d)) | lax.shift_right_logical(x, jnp.int32(32 - d))


def _threefry_bits(flat_i32):
    k1 = jnp.int32(0)
    k2 = jnp.int32(42)
    ks = [k1, k2, k1 ^ k2 ^ jnp.int32(0x1BD11BDA)]
    rot = ((13, 15, 26, 6), (17, 29, 16, 24))
    x0 = jnp.zeros_like(flat_i32) + ks[0]
    x1 = flat_i32 + ks[1]
    for i in range(5):
        for r in rot[i % 2]:
            x0 = x0 + x1
            x1 = _rol(x1, r)
            x1 = x0 ^ x1
        x0 = x0 + ks[(i + 1) % 3]
        x1 = x1 + ks[(i + 2) % 3] + jnp.int32(i + 1)
    return x0 ^ x1


def _gumbel_from_bits(bits):
    fb = lax.shift_right_logical(bits, jnp.int32(9)) | jnp.int32(0x3F800000)
    u = lax.bitcast_convert_type(fb, jnp.float32) - jnp.float32(1.0)
    u = jnp.maximum(jnp.float32(_TINY), u + jnp.float32(_TINY))
    return -jnp.log(-jnp.log(u))


def _body(x_ref, out_ref, vmax_ref, vidx_ref):
    j = pl.program_id(0)

    col = jax.lax.broadcasted_iota(jnp.int32, (B, BC), 1) + j * BC
    row = jax.lax.broadcasted_iota(jnp.int32, (B, BC), 0)
    flat = row * N + col

    g = _gumbel_from_bits(_threefry_bits(flat))
    score = jnp.log(x_ref[...]) + g
    score = jnp.where(col < N, score, jnp.float32(_NEG_INF))

    bmax = jnp.max(score, axis=1, keepdims=True)
    cand = jnp.where(score == bmax, col, jnp.int32(0x7FFFFFFF))
    bidx = jnp.min(cand, axis=1, keepdims=True)

    @pl.when(j == 0)
    def _init():
        vmax_ref[...] = bmax
        vidx_ref[...] = bidx

    @pl.when(j > 0)
    def _update():
        better = bmax > vmax_ref[...]
        vmax_ref[...] = jnp.where(better, bmax, vmax_ref[...])
        vidx_ref[...] = jnp.where(better, bidx, vidx_ref[...])

    @pl.when(j == NBLK - 1)
    def _emit():
        out_ref[...] = vidx_ref[...]


@jax.jit
def kernel(logits):
    out = pl.pallas_call(
        _body,
        grid=(NBLK,),
        in_specs=[pl.BlockSpec((B, BC), lambda j: (0, j))],
        out_specs=pl.BlockSpec((B, 1), lambda j: (0, 0)),
        out_shape=jax.ShapeDtypeStruct((B, 1), jnp.int32),
        scratch_shapes=[
            pltpu.VMEM((B, 1), jnp.float32),
            pltpu.VMEM((B, 1), jnp.int32),
        ],
        compiler_params=pltpu.CompilerParams(
            dimension_semantics=("arbitrary",),
        ),
    )(logits)
    return out.reshape(B)
